# baseline (speedup 1.0000x reference)
import jax
import jax.numpy as jnp
from jax import lax
from jax.experimental import pallas as pl
from jax.experimental.pallas import tpu as pltpu
from jax.experimental.pallas import tpu_sc as plsc

N_NODES = 10000
D = 128
N_EDGES = 320000

NC = 2
NS = 16
NW = NC * NS
E_PER_TILE = N_EDGES // NW
CH = 128
N_CHUNKS = 79
E_TILE_PAD = N_CHUNKS * CH
N_PAD = 10240
ROWS_PER_TILE = N_PAD // NS
DEG_W = 16

ROW_BLK = 1000

_sc_mesh = plsc.VectorSubcoreMesh(core_axis_name="c", subcore_axis_name="s")


def _deg_body(dst_hbm, ones_hbm, zeros_hbm, deg_out, idx_v, ones_v, shared_deg):
    c = lax.axis_index("c")
    s = lax.axis_index("s")
    base = (c * NS + s) * E_TILE_PAD
    row0 = s * ROWS_PER_TILE
    pltpu.sync_copy(zeros_hbm.at[pl.ds(row0, ROWS_PER_TILE)],
                    shared_deg.at[pl.ds(row0, ROWS_PER_TILE)])
    pltpu.sync_copy(ones_hbm, ones_v)
    plsc.subcore_barrier()

    def body(i, carry):
        pltpu.sync_copy(dst_hbm.at[pl.ds(base + i * CH, CH)], idx_v)
        pltpu.sync_copy(ones_v, shared_deg.at[idx_v], add=True)
        return carry

    lax.fori_loop(0, N_CHUNKS, body, 0)
    plsc.subcore_barrier()
    pltpu.sync_copy(shared_deg.at[pl.ds(row0, ROWS_PER_TILE)],
                    deg_out.at[c, pl.ds(row0, ROWS_PER_TILE)])


_deg_call = pl.kernel(
    _deg_body,
    out_type=jax.ShapeDtypeStruct((NC, N_PAD, D), jnp.float32),
    mesh=_sc_mesh,
    scratch_types=[
        pltpu.VMEM((CH,), jnp.int32),
        pltpu.VMEM((CH, D), jnp.float32),
        pltpu.VMEM_SHARED((N_PAD, D), jnp.float32),
    ],
)


def _prop_body(g_hbm, src_hbm, dst_hbm, zeros_hbm, s_out,
               idx_s, idx_d, rows_v, shared_s, sem):
    c = lax.axis_index("c")
    s = lax.axis_index("s")
    base = (c * NS + s) * E_TILE_PAD
    row0 = s * ROWS_PER_TILE
    pltpu.sync_copy(zeros_hbm.at[pl.ds(row0, ROWS_PER_TILE)],
                    shared_s.at[pl.ds(row0, ROWS_PER_TILE)])
    plsc.subcore_barrier()

    def body(i, carry):
        off = base + i * CH
        pltpu.sync_copy(src_hbm.at[pl.ds(off, CH)], idx_s)
        pltpu.sync_copy(dst_hbm.at[pl.ds(off, CH)], idx_d)
        pltpu.async_copy(g_hbm.at[idx_s], rows_v, sem).wait()
        pltpu.sync_copy(rows_v, shared_s.at[idx_d], add=True)
        return carry

    lax.fori_loop(0, N_CHUNKS, body, 0)
    plsc.subcore_barrier()
    pltpu.sync_copy(shared_s.at[pl.ds(row0, ROWS_PER_TILE)],
                    s_out.at[c, pl.ds(row0, ROWS_PER_TILE)])


_prop_call = pl.kernel(
    _prop_body,
    out_type=jax.ShapeDtypeStruct((NC, N_PAD, D), jnp.float32),
    mesh=_sc_mesh,
    scratch_types=[
        pltpu.VMEM((CH,), jnp.int32),
        pltpu.VMEM((CH,), jnp.int32),
        pltpu.VMEM((CH, D), jnp.float32),
        pltpu.VMEM_SHARED((N_PAD, D), jnp.float32),
        pltpu.SemaphoreType.DMA,
    ],
)


def _dinv(deg_ref):
    deg = deg_ref[0, :, :1] + deg_ref[1, :, :1] + 1.0
    return lax.rsqrt(deg)


def _mm_scale_kernel(x_ref, w_ref, deg_ref, o_ref):
    h = jnp.dot(x_ref[...], w_ref[...], preferred_element_type=jnp.float32)
    o_ref[...] = h * _dinv(deg_ref)


def _combine_mm_kernel(s_ref, g_ref, b_ref, w_ref, deg_ref, o_ref):
    dinv = _dinv(deg_ref)
    h = jnp.maximum(dinv * (s_ref[0] + s_ref[1] + g_ref[...])
                    + b_ref[...], 0.0)
    o_ref[...] = jnp.dot(h, w_ref[...],
                         preferred_element_type=jnp.float32) * dinv


def _final_kernel(s_ref, g_ref, b_ref, deg_ref, o_ref):
    o_ref[...] = (_dinv(deg_ref)
                  * (s_ref[0] + s_ref[1] + g_ref[...]) + b_ref[...])


def _row_blk(i):
    return (i, 0)


_nd_spec = pl.BlockSpec((ROW_BLK, D), _row_blk)
_s_spec = pl.BlockSpec((NC, ROW_BLK, D), lambda i: (0, i, 0))
_deg_spec = pl.BlockSpec((NC, ROW_BLK, D), lambda i: (0, i, 0))
_w_spec = pl.BlockSpec((D, D), lambda i: (0, 0))
_b_spec = pl.BlockSpec((1, D), lambda i: (0, 0))
_grid = (N_NODES // ROW_BLK,)
_out_nd = jax.ShapeDtypeStruct((N_NODES, D), jnp.float32)


def _mm_scale(x, w, deg):
    return pl.pallas_call(
        _mm_scale_kernel,
        grid=_grid,
        in_specs=[_nd_spec, _w_spec, _deg_spec],
        out_specs=_nd_spec,
        out_shape=_out_nd,
    )(x, w, deg)


def _combine_mm(s, g, b, w, deg):
    return pl.pallas_call(
        _combine_mm_kernel,
        grid=_grid,
        in_specs=[_s_spec, _nd_spec, _b_spec, _w_spec, _deg_spec],
        out_specs=_nd_spec,
        out_shape=_out_nd,
    )(s, g, b, w, deg)


def _final(s, g, b, deg):
    return pl.pallas_call(
        _final_kernel,
        grid=_grid,
        in_specs=[_s_spec, _nd_spec, _b_spec, _deg_spec],
        out_specs=_nd_spec,
        out_shape=_out_nd,
    )(s, g, b, deg)


@jax.jit
def _run(x, src, dst, W1, b1, W2, b2):
    ones_ch = jnp.ones((CH, D), jnp.float32)
    zeros_nd = jnp.zeros((N_PAD, D), jnp.float32)
    b1r = b1.reshape(1, D)
    b2r = b2.reshape(1, D)

    deg = _deg_call(dst, ones_ch, zeros_nd)
    g1 = _mm_scale(x, W1, deg)
    s1 = _prop_call(g1, src, dst, zeros_nd)
    g2 = _combine_mm(s1, g1, b1r, W2, deg)
    s2 = _prop_call(g2, src, dst, zeros_nd)
    return _final(s2, g2, b2r, deg)


def kernel(x, edge_index, W1, b1, W2, b2):
    ei = edge_index.astype(jnp.int32)
    pad = ((0, 0), (0, E_TILE_PAD - E_PER_TILE))
    src_p = jnp.pad(ei[0].reshape(NW, E_PER_TILE), pad).reshape(-1)
    dst_p = jnp.pad(ei[1].reshape(NW, E_PER_TILE), pad,
                    constant_values=N_PAD - 1).reshape(-1)
    return _run(x, src_p, dst_p, W1, b1, W2, b2)


# speedup vs baseline: 1.1489x; 1.1489x over previous
import jax
import jax.numpy as jnp
from jax import lax
from jax.experimental import pallas as pl
from jax.experimental.pallas import tpu as pltpu
from jax.experimental.pallas import tpu_sc as plsc

N_NODES = 10000
D = 128
N_EDGES = 320000

NC = 2
NS = 16
NW = NC * NS
E_PER_TILE = N_EDGES // NW
CH = 128
N_CHUNKS = 79
E_TILE_PAD = N_CHUNKS * CH
CH_P = 80
N_CHUNKS_P = E_PER_TILE // CH_P
N_PAD = 10240
ROWS_PER_TILE = N_PAD // NS
DEG_W = 16

ROW_BLK = 1000

_sc_mesh = plsc.VectorSubcoreMesh(core_axis_name="c", subcore_axis_name="s")


def _deg_body(dst_hbm, ones_hbm, zeros_hbm, deg_out, idx_v, ones_v, shared_deg):
    c = lax.axis_index("c")
    s = lax.axis_index("s")
    base = (c * NS + s) * E_TILE_PAD
    row0 = s * ROWS_PER_TILE
    pltpu.sync_copy(zeros_hbm.at[pl.ds(row0, ROWS_PER_TILE)],
                    shared_deg.at[pl.ds(row0, ROWS_PER_TILE)])
    pltpu.sync_copy(ones_hbm, ones_v)
    plsc.subcore_barrier()

    def body(i, carry):
        pltpu.sync_copy(dst_hbm.at[pl.ds(base + i * CH, CH)], idx_v)
        pltpu.sync_copy(ones_v, shared_deg.at[idx_v], add=True)
        return carry

    lax.fori_loop(0, N_CHUNKS, body, 0)
    plsc.subcore_barrier()
    pltpu.sync_copy(shared_deg.at[pl.ds(row0, ROWS_PER_TILE)],
                    deg_out.at[c, pl.ds(row0, ROWS_PER_TILE)])


_deg_call = pl.kernel(
    _deg_body,
    out_type=jax.ShapeDtypeStruct((NC, N_PAD, D), jnp.float32),
    mesh=_sc_mesh,
    scratch_types=[
        pltpu.VMEM((CH,), jnp.int32),
        pltpu.VMEM((CH, D), jnp.float32),
        pltpu.VMEM_SHARED((N_PAD, D), jnp.float32),
    ],
)


def _prop_body(g_hbm, src_hbm, dst_hbm, zeros_hbm, s_out,
               idx_s, idx_d, rows_v, shared_s, sem):
    c = lax.axis_index("c")
    s = lax.axis_index("s")
    base = (c * NS + s) * E_PER_TILE
    row0 = s * ROWS_PER_TILE
    pltpu.sync_copy(zeros_hbm.at[pl.ds(row0, ROWS_PER_TILE)],
                    shared_s.at[pl.ds(row0, ROWS_PER_TILE)])
    plsc.subcore_barrier()

    def body(i, carry):
        off = base + i * CH_P
        pltpu.sync_copy(src_hbm.at[pl.ds(off, CH_P)], idx_s)
        pltpu.sync_copy(dst_hbm.at[pl.ds(off, CH_P)], idx_d)
        pltpu.async_copy(g_hbm.at[idx_s], rows_v, sem).wait()
        pltpu.sync_copy(rows_v, shared_s.at[idx_d], add=True)
        return carry

    lax.fori_loop(0, N_CHUNKS_P, body, 0)
    plsc.subcore_barrier()
    pltpu.sync_copy(shared_s.at[pl.ds(row0, ROWS_PER_TILE)],
                    s_out.at[c, pl.ds(row0, ROWS_PER_TILE)])


_prop_call = pl.kernel(
    _prop_body,
    out_type=jax.ShapeDtypeStruct((NC, N_PAD, D), jnp.float32),
    mesh=_sc_mesh,
    scratch_types=[
        pltpu.VMEM((CH_P,), jnp.int32),
        pltpu.VMEM((CH_P,), jnp.int32),
        pltpu.VMEM((CH_P, D), jnp.float32),
        pltpu.VMEM_SHARED((N_PAD, D), jnp.float32),
        pltpu.SemaphoreType.DMA,
    ],
)


def _dinv(deg_ref):
    deg = deg_ref[0, :, :1] + deg_ref[1, :, :1] + 1.0
    return lax.rsqrt(deg)


def _mm_scale_kernel(x_ref, w_ref, deg_ref, o_ref):
    h = jnp.dot(x_ref[...], w_ref[...], preferred_element_type=jnp.float32)
    o_ref[...] = h * _dinv(deg_ref)


def _combine_mm_kernel(s_ref, g_ref, b_ref, w_ref, deg_ref, o_ref):
    dinv = _dinv(deg_ref)
    h = jnp.maximum(dinv * (s_ref[0] + s_ref[1] + g_ref[...])
                    + b_ref[...], 0.0)
    o_ref[...] = jnp.dot(h, w_ref[...],
                         preferred_element_type=jnp.float32) * dinv


def _final_kernel(s_ref, g_ref, b_ref, deg_ref, o_ref):
    o_ref[...] = (_dinv(deg_ref)
                  * (s_ref[0] + s_ref[1] + g_ref[...]) + b_ref[...])


def _row_blk(i):
    return (i, 0)


_nd_spec = pl.BlockSpec((ROW_BLK, D), _row_blk)
_s_spec = pl.BlockSpec((NC, ROW_BLK, D), lambda i: (0, i, 0))
_deg_spec = pl.BlockSpec((NC, ROW_BLK, D), lambda i: (0, i, 0))
_w_spec = pl.BlockSpec((D, D), lambda i: (0, 0))
_b_spec = pl.BlockSpec((1, D), lambda i: (0, 0))
_grid = (N_NODES // ROW_BLK,)
_out_nd = jax.ShapeDtypeStruct((N_NODES, D), jnp.float32)


def _mm_scale(x, w, deg):
    return pl.pallas_call(
        _mm_scale_kernel,
        grid=_grid,
        in_specs=[_nd_spec, _w_spec, _deg_spec],
        out_specs=_nd_spec,
        out_shape=_out_nd,
    )(x, w, deg)


def _combine_mm(s, g, b, w, deg):
    return pl.pallas_call(
        _combine_mm_kernel,
        grid=_grid,
        in_specs=[_s_spec, _nd_spec, _b_spec, _w_spec, _deg_spec],
        out_specs=_nd_spec,
        out_shape=_out_nd,
    )(s, g, b, w, deg)


def _final(s, g, b, deg):
    return pl.pallas_call(
        _final_kernel,
        grid=_grid,
        in_specs=[_s_spec, _nd_spec, _b_spec, _deg_spec],
        out_specs=_nd_spec,
        out_shape=_out_nd,
    )(s, g, b, deg)


@jax.jit
def _run(x, src, dst, dstp, W1, b1, W2, b2):
    ones_ch = jnp.ones((CH, D), jnp.float32)
    zeros_nd = jnp.zeros((N_PAD, D), jnp.float32)
    b1r = b1.reshape(1, D)
    b2r = b2.reshape(1, D)

    deg = _deg_call(dstp, ones_ch, zeros_nd)
    g1 = _mm_scale(x, W1, deg)
    s1 = _prop_call(g1, src, dst, zeros_nd)
    g2 = _combine_mm(s1, g1, b1r, W2, deg)
    s2 = _prop_call(g2, src, dst, zeros_nd)
    return _final(s2, g2, b2r, deg)


def kernel(x, edge_index, W1, b1, W2, b2):
    ei = edge_index.astype(jnp.int32)
    pad = ((0, 0), (0, E_TILE_PAD - E_PER_TILE))
    dst_p = jnp.pad(ei[1].reshape(NW, E_PER_TILE), pad,
                    constant_values=N_PAD - 1).reshape(-1)
    return _run(x, ei[0], ei[1], dst_p, W1, b1, W2, b2)


# speedup vs baseline: 1.3212x; 1.1499x over previous
import jax
import jax.numpy as jnp
from jax import lax
from jax.experimental import pallas as pl
from jax.experimental.pallas import tpu as pltpu
from jax.experimental.pallas import tpu_sc as plsc

N_NODES = 10000
D = 128
N_EDGES = 320000

NC = 2
NS = 16
NW = NC * NS
E_PER_TILE = N_EDGES // NW
CH = 128
N_CHUNKS = 79
E_TILE_PAD = N_CHUNKS * CH
CH_P = 80
N_CHUNKS_P = E_PER_TILE // CH_P
N_PAD = 10240
ROWS_PER_TILE = N_PAD // NS
DEG_W = 16

ROW_BLK = 1000

_sc_mesh = plsc.VectorSubcoreMesh(core_axis_name="c", subcore_axis_name="s")


def _deg_body(dst_hbm, ones_hbm, zeros_hbm, deg_out, idx_v, ones_v, shared_deg):
    c = lax.axis_index("c")
    s = lax.axis_index("s")
    base = (c * NS + s) * E_TILE_PAD
    row0 = s * ROWS_PER_TILE
    pltpu.sync_copy(zeros_hbm.at[pl.ds(row0, ROWS_PER_TILE)],
                    shared_deg.at[pl.ds(row0, ROWS_PER_TILE)])
    pltpu.sync_copy(ones_hbm, ones_v)
    plsc.subcore_barrier()

    def body(i, carry):
        pltpu.sync_copy(dst_hbm.at[pl.ds(base + i * CH, CH)], idx_v)
        pltpu.sync_copy(ones_v, shared_deg.at[idx_v], add=True)
        return carry

    lax.fori_loop(0, N_CHUNKS, body, 0)
    plsc.subcore_barrier()
    pltpu.sync_copy(shared_deg.at[pl.ds(row0, ROWS_PER_TILE)],
                    deg_out.at[c, pl.ds(row0, ROWS_PER_TILE)])


_deg_call = pl.kernel(
    _deg_body,
    out_type=jax.ShapeDtypeStruct((NC, N_PAD, D), jnp.float32),
    mesh=_sc_mesh,
    scratch_types=[
        pltpu.VMEM((CH,), jnp.int32),
        pltpu.VMEM((CH, D), jnp.float32),
        pltpu.VMEM_SHARED((N_PAD, D), jnp.float32),
    ],
)


def _prop_body(g_hbm, src_hbm, dst_hbm, zeros_hbm, s_out,
               idx_s, idx_d, rows_v, shared_s, sem):
    c = lax.axis_index("c")
    s = lax.axis_index("s")
    base = (c * NS + s) * E_PER_TILE
    row0 = s * ROWS_PER_TILE
    pltpu.sync_copy(zeros_hbm.at[pl.ds(row0, ROWS_PER_TILE)],
                    shared_s.at[pl.ds(row0, ROWS_PER_TILE)])
    pltpu.sync_copy(src_hbm.at[pl.ds(base, E_PER_TILE)], idx_s)
    plsc.subcore_barrier()

    def body(i, carry):
        pltpu.sync_copy(dst_hbm.at[pl.ds(base + i * CH_P, CH_P)], idx_d)
        pltpu.async_copy(g_hbm.at[idx_s.at[pl.ds(i * CH_P, CH_P)]],
                         rows_v, sem).wait()
        pltpu.sync_copy(rows_v, shared_s.at[idx_d], add=True)
        return carry

    lax.fori_loop(0, N_CHUNKS_P, body, 0)
    plsc.subcore_barrier()
    pltpu.sync_copy(shared_s.at[pl.ds(row0, ROWS_PER_TILE)],
                    s_out.at[c, pl.ds(row0, ROWS_PER_TILE)])


_prop_call = pl.kernel(
    _prop_body,
    out_type=jax.ShapeDtypeStruct((NC, N_PAD, D), jnp.float32),
    mesh=_sc_mesh,
    scratch_types=[
        pltpu.VMEM((E_PER_TILE,), jnp.int32),
        pltpu.VMEM((CH_P,), jnp.int32),
        pltpu.VMEM((CH_P, D), jnp.float32),
        pltpu.VMEM_SHARED((N_PAD, D), jnp.float32),
        pltpu.SemaphoreType.DMA,
    ],
)


def _dinv(deg_ref):
    deg = deg_ref[0, :, :1] + deg_ref[1, :, :1] + 1.0
    return lax.rsqrt(deg)


def _mm_scale_kernel(x_ref, w_ref, deg_ref, o_ref):
    h = jnp.dot(x_ref[...], w_ref[...], preferred_element_type=jnp.float32)
    o_ref[...] = h * _dinv(deg_ref)


def _combine_mm_kernel(s_ref, g_ref, b_ref, w_ref, deg_ref, o_ref):
    dinv = _dinv(deg_ref)
    h = jnp.maximum(dinv * (s_ref[0] + s_ref[1] + g_ref[...])
                    + b_ref[...], 0.0)
    o_ref[...] = jnp.dot(h, w_ref[...],
                         preferred_element_type=jnp.float32) * dinv


def _final_kernel(s_ref, g_ref, b_ref, deg_ref, o_ref):
    o_ref[...] = (_dinv(deg_ref)
                  * (s_ref[0] + s_ref[1] + g_ref[...]) + b_ref[...])


def _row_blk(i):
    return (i, 0)


_nd_spec = pl.BlockSpec((ROW_BLK, D), _row_blk)
_s_spec = pl.BlockSpec((NC, ROW_BLK, D), lambda i: (0, i, 0))
_deg_spec = pl.BlockSpec((NC, ROW_BLK, D), lambda i: (0, i, 0))
_w_spec = pl.BlockSpec((D, D), lambda i: (0, 0))
_b_spec = pl.BlockSpec((1, D), lambda i: (0, 0))
_grid = (N_NODES // ROW_BLK,)
_out_nd = jax.ShapeDtypeStruct((N_NODES, D), jnp.float32)


def _mm_scale(x, w, deg):
    return pl.pallas_call(
        _mm_scale_kernel,
        grid=_grid,
        in_specs=[_nd_spec, _w_spec, _deg_spec],
        out_specs=_nd_spec,
        out_shape=_out_nd,
    )(x, w, deg)


def _combine_mm(s, g, b, w, deg):
    return pl.pallas_call(
        _combine_mm_kernel,
        grid=_grid,
        in_specs=[_s_spec, _nd_spec, _b_spec, _w_spec, _deg_spec],
        out_specs=_nd_spec,
        out_shape=_out_nd,
    )(s, g, b, w, deg)


def _final(s, g, b, deg):
    return pl.pallas_call(
        _final_kernel,
        grid=_grid,
        in_specs=[_s_spec, _nd_spec, _b_spec, _deg_spec],
        out_specs=_nd_spec,
        out_shape=_out_nd,
    )(s, g, b, deg)


@jax.jit
def _run(x, src, dst, dstp, W1, b1, W2, b2):
    ones_ch = jnp.ones((CH, D), jnp.float32)
    zeros_nd = jnp.zeros((N_PAD, D), jnp.float32)
    b1r = b1.reshape(1, D)
    b2r = b2.reshape(1, D)

    deg = _deg_call(dstp, ones_ch, zeros_nd)
    g1 = _mm_scale(x, W1, deg)
    s1 = _prop_call(g1, src, dst, zeros_nd)
    g2 = _combine_mm(s1, g1, b1r, W2, deg)
    s2 = _prop_call(g2, src, dst, zeros_nd)
    return _final(s2, g2, b2r, deg)


def kernel(x, edge_index, W1, b1, W2, b2):
    ei = edge_index.astype(jnp.int32)
    pad = ((0, 0), (0, E_TILE_PAD - E_PER_TILE))
    dst_p = jnp.pad(ei[1].reshape(NW, E_PER_TILE), pad,
                    constant_values=N_PAD - 1).reshape(-1)
    return _run(x, ei[0], ei[1], dst_p, W1, b1, W2, b2)
